# Initial kernel scaffold; baseline (speedup 1.0000x reference)
import jax
import jax.numpy as jnp
from jax.experimental import pallas as pl


def kernel(x, Wr, br, We, be):
    raise NotImplementedError("write your pallas kernel here")


# speedup vs baseline: 1.3497x; 1.3497x over previous
import functools

import jax
import jax.numpy as jnp
from jax.experimental import pallas as pl
from jax.experimental.pallas import tpu as pltpu

NUM_EXPERTS = 8
HIDDEN = 1024
MT = 512


def _moe_body(x_ref, wr_ref, br_ref, we_ref, be_ref, out_ref, coeff_ref):
    e = pl.program_id(1)

    @pl.when(e == 0)
    def _():
        xb = x_ref[...].astype(jnp.float32)
        wrb = wr_ref[...].astype(jnp.float32)
        scores = jax.lax.dot_general(
            xb, wrb, (((1,), (1,)), ((), ())),
            preferred_element_type=jnp.float32,
        ) + br_ref[...]
        m = jnp.max(scores, axis=1, keepdims=True)
        p = jnp.exp(scores - m)
        p = p / jnp.sum(p, axis=1, keepdims=True)
        iota = jax.lax.broadcasted_iota(jnp.int32, p.shape, 1)
        m0 = jnp.max(p, axis=1, keepdims=True)
        a0 = jnp.min(jnp.where(p == m0, iota, NUM_EXPERTS), axis=1,
                     keepdims=True)
        p1m = jnp.where(iota == a0, -1.0, p)
        m1 = jnp.max(p1m, axis=1, keepdims=True)
        a1 = jnp.min(jnp.where(p1m == m1, iota, NUM_EXPERTS), axis=1,
                     keepdims=True)
        wsum = m0 + m1
        coeff_ref[...] = (m0 * (iota == a0) + m1 * (iota == a1)) / wsum

    xb = x_ref[...].astype(jnp.float32)
    web = we_ref[0].astype(jnp.float32)
    y = jax.lax.dot_general(
        xb, web, (((1,), (1,)), ((), ())),
        preferred_element_type=jnp.float32,
    ) + be_ref[0, 0]
    iota = jax.lax.broadcasted_iota(jnp.int32, coeff_ref.shape, 1)
    ce = jnp.sum(jnp.where(iota == e, coeff_ref[...], 0.0), axis=1,
                 keepdims=True)

    @pl.when(e == 0)
    def _():
        out_ref[...] = ce * y

    @pl.when(e != 0)
    def _():
        out_ref[...] += ce * y


def kernel(x, Wr, br, We, be):
    B, S, H = x.shape
    M = B * S
    xf = x.reshape(M, H)
    br2 = br.reshape(1, NUM_EXPERTS)
    be3 = be.reshape(NUM_EXPERTS, 1, H)
    grid = (M // MT, NUM_EXPERTS)
    out = pl.pallas_call(
        _moe_body,
        grid=grid,
        in_specs=[
            pl.BlockSpec((MT, H), lambda t, e: (t, 0)),
            pl.BlockSpec((NUM_EXPERTS, H), lambda t, e: (0, 0)),
            pl.BlockSpec((1, NUM_EXPERTS), lambda t, e: (0, 0)),
            pl.BlockSpec((1, H, H), lambda t, e: (e, 0, 0)),
            pl.BlockSpec((1, 1, H), lambda t, e: (e, 0, 0)),
        ],
        out_specs=pl.BlockSpec((MT, H), lambda t, e: (t, 0)),
        out_shape=jax.ShapeDtypeStruct((M, H), jnp.float32),
        scratch_shapes=[pltpu.VMEM((MT, NUM_EXPERTS), jnp.float32)],
    )(xf, Wr, br2, We, be3)
    return out.reshape(B, S, H)


# speedup vs baseline: 2.3443x; 1.7370x over previous
import jax
import jax.numpy as jnp
from jax.experimental import pallas as pl
from jax.experimental.pallas import tpu as pltpu

NUM_EXPERTS = 8
HIDDEN = 1024
MT = 1024


def _moe_body(x_ref, wr_ref, br_ref, we_ref, be_ref, out_ref,
              coeff_ref, xb_ref, web_ref):
    xf = x_ref[...]
    scores = jax.lax.dot_general(
        xf, wr_ref[...], (((1,), (1,)), ((), ())),
        preferred_element_type=jnp.float32,
    ) + br_ref[...]
    m = jnp.max(scores, axis=1, keepdims=True)
    p = jnp.exp(scores - m)
    p = p / jnp.sum(p, axis=1, keepdims=True)
    iota = jax.lax.broadcasted_iota(jnp.int32, p.shape, 1)
    m0 = jnp.max(p, axis=1, keepdims=True)
    a0 = jnp.min(jnp.where(p == m0, iota, NUM_EXPERTS), axis=1, keepdims=True)
    p1m = jnp.where(iota == a0, -1.0, p)
    m1 = jnp.max(p1m, axis=1, keepdims=True)
    a1 = jnp.min(jnp.where(p1m == m1, iota, NUM_EXPERTS), axis=1,
                 keepdims=True)
    wsum = m0 + m1
    coeff_ref[...] = (m0 * (iota == a0) + m1 * (iota == a1)) / wsum

    xb_ref[...] = xf.astype(jnp.bfloat16)
    for e in range(NUM_EXPERTS):
        web_ref[...] = we_ref[e].astype(jnp.bfloat16)
        y = jax.lax.dot_general(
            xb_ref[...], web_ref[...], (((1,), (1,)), ((), ())),
            preferred_element_type=jnp.float32,
        ) + be_ref[e, 0]
        ce = coeff_ref[:, e:e + 1]
        if e == 0:
            out_ref[...] = ce * y
        else:
            out_ref[...] += ce * y


def kernel(x, Wr, br, We, be):
    B, S, H = x.shape
    M = B * S
    xf = x.reshape(M, H)
    br2 = br.reshape(1, NUM_EXPERTS)
    be3 = be.reshape(NUM_EXPERTS, 1, H)
    grid = (M // MT,)
    out = pl.pallas_call(
        _moe_body,
        grid=grid,
        in_specs=[
            pl.BlockSpec((MT, H), lambda t: (t, 0)),
            pl.BlockSpec((NUM_EXPERTS, H), lambda t: (0, 0)),
            pl.BlockSpec((1, NUM_EXPERTS), lambda t: (0, 0)),
            pl.BlockSpec((NUM_EXPERTS, H, H), lambda t: (0, 0, 0)),
            pl.BlockSpec((NUM_EXPERTS, 1, H), lambda t: (0, 0, 0)),
        ],
        out_specs=pl.BlockSpec((MT, H), lambda t: (t, 0)),
        out_shape=jax.ShapeDtypeStruct((M, H), jnp.float32),
        scratch_shapes=[
            pltpu.VMEM((MT, NUM_EXPERTS), jnp.float32),
            pltpu.VMEM((MT, HIDDEN), jnp.bfloat16),
            pltpu.VMEM((HIDDEN, HIDDEN), jnp.bfloat16),
        ],
    )(xf, Wr, br2, We, be3)
    return out.reshape(B, S, H)
